# Initial kernel scaffold; baseline (speedup 1.0000x reference)
import jax
import jax.numpy as jnp
from jax.experimental import pallas as pl


def kernel(feature, edge_index, W_halt, b_halt):
    raise NotImplementedError("write your pallas kernel here")


# speedup vs baseline: 3.8984x; 3.8984x over previous
import functools

import jax
import jax.numpy as jnp
from jax import lax
from jax.experimental import pallas as pl
from jax.experimental.pallas import tpu as pltpu
from jax.experimental.pallas import tpu_sc as plsc

N_NODES = 10000
N_EDGES = 320000
D_FEAT = 128
NITER = 10

NC, NS = 2, 16
NTILES = NC * NS
EDGES_PER_TILE = N_EDGES // NTILES
EDGES_PER_SUB = N_EDGES // NS
CHUNK = 80
N_PAD = 10240
ROWS_PER_TILE = N_PAD // NS
ZCH = 128
DEG_W = 16

_mesh = plsc.VectorSubcoreMesh(
    core_axis_name="c", subcore_axis_name="s", num_cores=NC, num_subcores=NS)


def _zero_fill(buf, rows, width):
    @pl.loop(0, rows)
    def _(i):
        @pl.loop(0, width // 16)
        def _(j):
            buf.at[pl.ds(i, 1), pl.ds(j * 16, 16)][...] = (
                jnp.zeros((1, 16), jnp.float32))


@functools.partial(
    pl.kernel,
    out_type=jax.ShapeDtypeStruct((NC, N_PAD, 128), jnp.float32),
    mesh=_mesh,
    scratch_types=[
        pltpu.VMEM_SHARED((N_PAD, 128), jnp.float32),
        pltpu.VMEM((CHUNK,), jnp.int32),
        pltpu.VMEM((CHUNK, 128), jnp.float32),
        pltpu.VMEM((ZCH, 128), jnp.float32),
    ],
)
def _deg_kernel(edges_hbm, deg_hbm, deg_sh, idx_v, ones_v, zbuf):
    cid = lax.axis_index("c")
    sid = lax.axis_index("s")
    _zero_fill(zbuf, ZCH, 128)

    @pl.loop(0, CHUNK)
    def _(i):
        @pl.loop(0, 8)
        def _(j):
            ones_v.at[pl.ds(i, 1), pl.ds(j * 16, 16)][...] = (
                jnp.ones((1, 16), jnp.float32))

    @pl.loop(0, ROWS_PER_TILE // ZCH)
    def _(k):
        pltpu.sync_copy(zbuf,
                        deg_sh.at[pl.ds(sid * ROWS_PER_TILE + k * ZCH, ZCH)])

    plsc.subcore_barrier()

    @pl.loop(0, EDGES_PER_SUB // CHUNK)
    def _(blk):
        off = cid * N_EDGES + sid * EDGES_PER_SUB + blk * CHUNK
        pltpu.sync_copy(edges_hbm.at[pl.ds(off, CHUNK)], idx_v)
        pltpu.sync_copy(ones_v, deg_sh.at[idx_v], add=True)

    plsc.subcore_barrier()
    pltpu.sync_copy(
        deg_sh.at[pl.ds(sid * ROWS_PER_TILE, ROWS_PER_TILE)],
        deg_hbm.at[cid, pl.ds(sid * ROWS_PER_TILE, ROWS_PER_TILE)])


@functools.partial(
    pl.kernel,
    out_type=jax.ShapeDtypeStruct((NC, N_PAD, D_FEAT), jnp.float32),
    mesh=_mesh,
    scratch_types=[
        pltpu.VMEM_SHARED((N_PAD, D_FEAT), jnp.float32),
        pltpu.VMEM((CHUNK,), jnp.int32),
        pltpu.VMEM((CHUNK,), jnp.int32),
        pltpu.VMEM((CHUNK, D_FEAT), jnp.float32),
        pltpu.VMEM((ZCH, D_FEAT), jnp.float32),
        pltpu.SemaphoreType.DMA,
    ],
)
def _agg_kernel(fs_hbm, edges_hbm, acc_hbm, acc_sh, sidx_v, didx_v, rows_v,
                zbuf, sem):
    cid = lax.axis_index("c")
    sid = lax.axis_index("s")
    wid = cid * NS + sid
    _zero_fill(zbuf, ZCH, D_FEAT)

    @pl.loop(0, ROWS_PER_TILE // ZCH)
    def _(k):
        pltpu.sync_copy(zbuf,
                        acc_sh.at[pl.ds(sid * ROWS_PER_TILE + k * ZCH, ZCH)])

    plsc.subcore_barrier()

    base = wid * EDGES_PER_TILE

    @pl.loop(0, EDGES_PER_TILE // CHUNK)
    def _(blk):
        off = base + blk * CHUNK
        pltpu.sync_copy(edges_hbm.at[pl.ds(off, CHUNK)], sidx_v)
        pltpu.sync_copy(edges_hbm.at[pl.ds(N_EDGES + off, CHUNK)], didx_v)
        pltpu.async_copy(fs_hbm.at[sidx_v], rows_v, sem).wait()
        pltpu.sync_copy(rows_v, acc_sh.at[didx_v], add=True)

    plsc.subcore_barrier()
    pltpu.sync_copy(
        acc_sh.at[pl.ds(sid * ROWS_PER_TILE, ROWS_PER_TILE)],
        acc_hbm.at[cid, pl.ds(sid * ROWS_PER_TILE, ROWS_PER_TILE)])


def _fs_body(feat_ref, deg_ref, fs_ref):
    sn = lax.rsqrt(jnp.maximum(deg_ref[0:N_NODES, 0:1], 1.0))
    fs_ref[...] = feat_ref[...] * sn


_fs_call = pl.pallas_call(
    _fs_body,
    out_shape=jax.ShapeDtypeStruct((N_NODES, D_FEAT), jnp.float32),
)


FIN_B = 1000


def _fin_body(pp_ref, deg_ref, feat_ref, w_ref, b_ref, x_ref, steps_ref,
              rem_ref):
    dn = lax.rsqrt(jnp.maximum(deg_ref[:, 0:1], 1.0))
    prop = (pp_ref[0] + pp_ref[1]) * dn
    lg = jnp.dot(prop, w_ref[...])[:, 0:1] + b_ref[...]
    h = jax.nn.sigmoid(lg)
    steps = jnp.ones((FIN_B, 1), jnp.float32)
    sum_h = jnp.zeros((FIN_B, 1), jnp.float32)
    cont = jnp.ones((FIN_B, 1), dtype=jnp.bool_)
    acoef = jnp.zeros((FIN_B, 1), jnp.float32)
    bcoef = jnp.zeros((FIN_B, 1), jnp.float32)
    for k in range(NITER):
        cf = cont.astype(jnp.float32)
        pm = (sum_h + h < 0.99) & cont
        pf = pm.astype(jnp.float32)
        steps = steps + pf
        sum_h = sum_h + pf * h
        cond = pm & (steps < float(NITER))
        p = jnp.where(cond, sum_h, 1.0 - sum_h)
        if k == 0:
            acoef = acoef + p * cf
            bcoef = bcoef + (1.0 - p) * cf
        else:
            acoef = acoef + cf
        cont = cont & pm
    x_ref[...] = (acoef * prop + bcoef * feat_ref[...]) / steps
    steps_ref[...] = steps
    rem_ref[...] = 1.0 - sum_h


_fin_call = pl.pallas_call(
    _fin_body,
    grid=(N_NODES // FIN_B,),
    in_specs=[
        pl.BlockSpec((2, FIN_B, D_FEAT), lambda i: (0, i, 0)),
        pl.BlockSpec((FIN_B, 128), lambda i: (i, 0)),
        pl.BlockSpec((FIN_B, D_FEAT), lambda i: (i, 0)),
        pl.BlockSpec((D_FEAT, D_FEAT), lambda i: (0, 0)),
        pl.BlockSpec((1, 1), lambda i: (0, 0)),
    ],
    out_specs=(
        pl.BlockSpec((FIN_B, D_FEAT), lambda i: (i, 0)),
        pl.BlockSpec((FIN_B, 1), lambda i: (i, 0)),
        pl.BlockSpec((FIN_B, 1), lambda i: (i, 0)),
    ),
    out_shape=(
        jax.ShapeDtypeStruct((N_NODES, D_FEAT), jnp.float32),
        jax.ShapeDtypeStruct((N_NODES, 1), jnp.float32),
        jax.ShapeDtypeStruct((N_NODES, 1), jnp.float32),
    ),
)


@jax.jit
def kernel(feature, edge_index, W_halt, b_halt):
    edges_flat = edge_index.reshape(-1)
    degs = _deg_kernel(edges_flat)
    fs = _fs_call(feature, degs[0])
    pp = _agg_kernel(fs, edges_flat)
    w128 = jnp.zeros((D_FEAT, D_FEAT), jnp.float32).at[:, 0].set(W_halt[0])
    x, steps2, rem2 = _fin_call(pp, degs[1], feature, w128,
                                b_halt.reshape(1, 1))
    return (x, steps2.reshape(-1), rem2.reshape(-1))


# speedup vs baseline: 8.7126x; 2.2349x over previous
import dataclasses
import functools

import jax
import jax.numpy as jnp
from jax import lax
from jax.experimental import pallas as pl
from jax.experimental.pallas import tpu as pltpu
from jax.experimental.pallas import tpu_sc as plsc

N_NODES = 10000
N_EDGES = 320000
D_FEAT = 128
NITER = 10

NC, NS = 2, 16
NTILES = NC * NS
EDGES_PER_TILE = N_EDGES // NTILES
EDGES_PER_SUB = N_EDGES // NS
CHUNK = 80
N_PAD = 10240
ROWS_PER_TILE = N_PAD // NS
ZCH = 128
DEG_W = 16

_mesh = plsc.VectorSubcoreMesh(
    core_axis_name="c", subcore_axis_name="s", num_cores=NC, num_subcores=NS)


def _zero_fill(buf, rows, width):
    @pl.loop(0, rows)
    def _(i):
        @pl.loop(0, width // 16)
        def _(j):
            buf.at[pl.ds(i, 1), pl.ds(j * 16, 16)][...] = (
                jnp.zeros((1, 16), jnp.float32))


CH_H = 2000
N_HCH = EDGES_PER_SUB // CH_H

_cp_no_layout = pltpu.CompilerParams()
if "needs_layout_passes" in pltpu.CompilerParams.__dataclass_fields__:
    _cp_no_layout = dataclasses.replace(_cp_no_layout,
                                        needs_layout_passes=False)


@functools.partial(
    pl.kernel,
    compiler_params=_cp_no_layout,
    out_type=jax.ShapeDtypeStruct((2 * N_PAD,), jnp.float32),
    mesh=_mesh,
    scratch_types=[
        pltpu.VMEM_SHARED((NS, N_PAD), jnp.float32),
        pltpu.VMEM((N_PAD,), jnp.float32),
        pltpu.VMEM((CH_H,), jnp.int32),
        pltpu.VMEM((NS, ROWS_PER_TILE), jnp.float32),
        pltpu.VMEM((ROWS_PER_TILE,), jnp.float32),
    ],
)
def _deg_kernel(edges_hbm, deg_hbm, stg_sh, hist_v, idx_v, rbuf, obuf):
    cid = lax.axis_index("c")
    sid = lax.axis_index("s")
    ones16 = jnp.ones((16,), jnp.float32)

    @pl.loop(0, N_PAD // 16)
    def _(i):
        hist_v.at[pl.ds(i * 16, 16)][...] = jnp.zeros((16,), jnp.float32)

    @pl.loop(0, N_HCH)
    def _(k):
        off = cid * N_EDGES + sid * EDGES_PER_SUB + k * CH_H
        pltpu.sync_copy(edges_hbm.at[pl.ds(off, CH_H)], idx_v)

        @pl.loop(0, CH_H // 16)
        def _(j):
            iv = idx_v[pl.ds(j * 16, 16)]
            plsc.addupdate_scatter(hist_v, [iv], ones16)

    pltpu.sync_copy(hist_v, stg_sh.at[sid])
    plsc.subcore_barrier()

    pltpu.sync_copy(
        stg_sh.at[pl.ds(0, NS), pl.ds(sid * ROWS_PER_TILE, ROWS_PER_TILE)],
        rbuf)

    @pl.loop(0, ROWS_PER_TILE // 16)
    def _(j):
        acc = rbuf[0, pl.ds(j * 16, 16)]
        def add_row(t, a):
            return a + rbuf[t, pl.ds(j * 16, 16)]
        acc = lax.fori_loop(1, NS, add_row, acc)
        obuf.at[pl.ds(j * 16, 16)][...] = acc

    pltpu.sync_copy(
        obuf, deg_hbm.at[pl.ds(cid * N_PAD + sid * ROWS_PER_TILE,
                               ROWS_PER_TILE)])


N_BLKS = EDGES_PER_TILE // CHUNK


@functools.partial(
    pl.kernel,
    out_type=jax.ShapeDtypeStruct((NC, N_PAD, D_FEAT), jnp.float32),
    mesh=_mesh,
    scratch_types=[
        pltpu.VMEM_SHARED((N_PAD, D_FEAT), jnp.float32),
        pltpu.VMEM((CHUNK,), jnp.int32),
        pltpu.VMEM((CHUNK,), jnp.int32),
        pltpu.VMEM((CHUNK,), jnp.int32),
        pltpu.VMEM((CHUNK,), jnp.int32),
        pltpu.VMEM((CHUNK, D_FEAT), jnp.float32),
        pltpu.VMEM((CHUNK, D_FEAT), jnp.float32),
        pltpu.VMEM((ZCH, D_FEAT), jnp.float32),
        pltpu.SemaphoreType.DMA,
        pltpu.SemaphoreType.DMA,
    ],
)
def _agg_kernel(fs_hbm, edges_hbm, acc_hbm, acc_sh, sidx_a, didx_a, sidx_b,
                didx_b, rows_a, rows_b, zbuf, sem_a, sem_b):
    cid = lax.axis_index("c")
    sid = lax.axis_index("s")
    wid = cid * NS + sid
    _zero_fill(zbuf, ZCH, D_FEAT)

    @pl.loop(0, ROWS_PER_TILE // ZCH)
    def _(k):
        pltpu.sync_copy(zbuf,
                        acc_sh.at[pl.ds(sid * ROWS_PER_TILE + k * ZCH, ZCH)])

    plsc.subcore_barrier()

    base = wid * EDGES_PER_TILE

    def load_idx(blk, sidx, didx):
        off = base + blk * CHUNK
        pltpu.sync_copy(edges_hbm.at[pl.ds(off, CHUNK)], sidx)
        pltpu.sync_copy(edges_hbm.at[pl.ds(N_EDGES + off, CHUNK)], didx)

    load_idx(0, sidx_a, didx_a)
    pltpu.async_copy(fs_hbm.at[sidx_a], rows_a, sem_a)

    @pl.loop(0, (N_BLKS - 1) // 2)
    def _(k):
        load_idx(2 * k + 1, sidx_b, didx_b)
        pltpu.async_copy(fs_hbm.at[sidx_b], rows_b, sem_b)
        pltpu.make_async_copy(fs_hbm.at[sidx_a], rows_a, sem_a).wait()
        pltpu.sync_copy(rows_a, acc_sh.at[didx_a], add=True)
        load_idx(2 * k + 2, sidx_a, didx_a)
        pltpu.async_copy(fs_hbm.at[sidx_a], rows_a, sem_a)
        pltpu.make_async_copy(fs_hbm.at[sidx_b], rows_b, sem_b).wait()
        pltpu.sync_copy(rows_b, acc_sh.at[didx_b], add=True)

    pltpu.make_async_copy(fs_hbm.at[sidx_a], rows_a, sem_a).wait()
    pltpu.sync_copy(rows_a, acc_sh.at[didx_a], add=True)

    plsc.subcore_barrier()
    pltpu.sync_copy(
        acc_sh.at[pl.ds(sid * ROWS_PER_TILE, ROWS_PER_TILE)],
        acc_hbm.at[cid, pl.ds(sid * ROWS_PER_TILE, ROWS_PER_TILE)])


def _fs_body(feat_ref, deg_ref, fs_ref):
    sn = lax.rsqrt(jnp.maximum(deg_ref[...], 1.0))
    fs_ref[...] = feat_ref[...] * sn


_fs_call = pl.pallas_call(
    _fs_body,
    out_shape=jax.ShapeDtypeStruct((N_NODES, D_FEAT), jnp.float32),
)


FIN_B = 1000


def _fin_body(pp_ref, deg_ref, feat_ref, w_ref, b_ref, x_ref, steps_ref,
              rem_ref):
    dn = lax.rsqrt(jnp.maximum(deg_ref[...], 1.0))
    prop = (pp_ref[0] + pp_ref[1]) * dn
    lg = jnp.dot(prop, w_ref[...])[:, 0:1] + b_ref[...]
    h = jax.nn.sigmoid(lg)
    steps = jnp.ones((FIN_B, 1), jnp.float32)
    sum_h = jnp.zeros((FIN_B, 1), jnp.float32)
    cont = jnp.ones((FIN_B, 1), dtype=jnp.bool_)
    acoef = jnp.zeros((FIN_B, 1), jnp.float32)
    bcoef = jnp.zeros((FIN_B, 1), jnp.float32)
    for k in range(NITER):
        cf = cont.astype(jnp.float32)
        pm = (sum_h + h < 0.99) & cont
        pf = pm.astype(jnp.float32)
        steps = steps + pf
        sum_h = sum_h + pf * h
        cond = pm & (steps < float(NITER))
        p = jnp.where(cond, sum_h, 1.0 - sum_h)
        if k == 0:
            acoef = acoef + p * cf
            bcoef = bcoef + (1.0 - p) * cf
        else:
            acoef = acoef + cf
        cont = cont & pm
    x_ref[...] = (acoef * prop + bcoef * feat_ref[...]) / steps
    steps_ref[...] = steps
    rem_ref[...] = 1.0 - sum_h


_fin_call = pl.pallas_call(
    _fin_body,
    grid=(N_NODES // FIN_B,),
    in_specs=[
        pl.BlockSpec((2, FIN_B, D_FEAT), lambda i: (0, i, 0)),
        pl.BlockSpec((FIN_B, 1), lambda i: (i, 0)),
        pl.BlockSpec((FIN_B, D_FEAT), lambda i: (i, 0)),
        pl.BlockSpec((D_FEAT, D_FEAT), lambda i: (0, 0)),
        pl.BlockSpec((1, 1), lambda i: (0, 0)),
    ],
    out_specs=(
        pl.BlockSpec((FIN_B, D_FEAT), lambda i: (i, 0)),
        pl.BlockSpec((FIN_B, 1), lambda i: (i, 0)),
        pl.BlockSpec((FIN_B, 1), lambda i: (i, 0)),
    ),
    out_shape=(
        jax.ShapeDtypeStruct((N_NODES, D_FEAT), jnp.float32),
        jax.ShapeDtypeStruct((N_NODES, 1), jnp.float32),
        jax.ShapeDtypeStruct((N_NODES, 1), jnp.float32),
    ),
)


@jax.jit
def kernel(feature, edge_index, W_halt, b_halt):
    edges_flat = edge_index.reshape(-1)
    degs = _deg_kernel(edges_flat)
    out_deg = degs[0:N_NODES, None]
    in_deg = degs[N_PAD:N_PAD + N_NODES, None]
    fs = _fs_call(feature, out_deg)
    pp = _agg_kernel(fs, edges_flat)
    w128 = jnp.zeros((D_FEAT, D_FEAT), jnp.float32).at[:, 0].set(W_halt[0])
    x, steps2, rem2 = _fin_call(pp, in_deg, feature, w128,
                                b_halt.reshape(1, 1))
    return (x, steps2.reshape(-1), rem2.reshape(-1))


# speedup vs baseline: 10.9035x; 1.2515x over previous
import dataclasses
import functools

import jax
import jax.numpy as jnp
from jax import lax
from jax.experimental import pallas as pl
from jax.experimental.pallas import tpu as pltpu
from jax.experimental.pallas import tpu_sc as plsc

N_NODES = 10000
N_EDGES = 320000
D_FEAT = 128
NITER = 10

NC, NS = 2, 16
NTILES = NC * NS
EDGES_PER_TILE = N_EDGES // NTILES
EDGES_PER_SUB = N_EDGES // NS
CHUNK = 80
N_PAD = 10240
ROWS_PER_TILE = N_PAD // NS
ZCH = 128
DEG_W = 16

_mesh = plsc.VectorSubcoreMesh(
    core_axis_name="c", subcore_axis_name="s", num_cores=NC, num_subcores=NS)


def _zero_fill(buf, rows, width):
    @pl.loop(0, rows)
    def _(i):
        @pl.loop(0, width // 16)
        def _(j):
            buf.at[pl.ds(i, 1), pl.ds(j * 16, 16)][...] = (
                jnp.zeros((1, 16), jnp.float32))


CH_H = 2000
N_HCH = EDGES_PER_SUB // CH_H

_cp_no_layout = pltpu.CompilerParams()
if "needs_layout_passes" in pltpu.CompilerParams.__dataclass_fields__:
    _cp_no_layout = dataclasses.replace(_cp_no_layout,
                                        needs_layout_passes=False)


@functools.partial(
    pl.kernel,
    compiler_params=_cp_no_layout,
    out_type=jax.ShapeDtypeStruct((2 * N_PAD,), jnp.float32),
    mesh=_mesh,
    scratch_types=[
        pltpu.VMEM_SHARED((NS, N_PAD), jnp.float32),
        pltpu.VMEM((N_PAD,), jnp.float32),
        pltpu.VMEM((CH_H,), jnp.int32),
        pltpu.VMEM((NS, ROWS_PER_TILE), jnp.float32),
        pltpu.VMEM((ROWS_PER_TILE,), jnp.float32),
    ],
)
def _deg_kernel(edges_hbm, deg_hbm, stg_sh, hist_v, idx_v, rbuf, obuf):
    cid = lax.axis_index("c")
    sid = lax.axis_index("s")
    ones16 = jnp.ones((16,), jnp.float32)

    @pl.loop(0, N_PAD // 16)
    def _(i):
        hist_v.at[pl.ds(i * 16, 16)][...] = jnp.zeros((16,), jnp.float32)

    @pl.loop(0, N_HCH)
    def _(k):
        off = cid * N_EDGES + sid * EDGES_PER_SUB + k * CH_H
        pltpu.sync_copy(edges_hbm.at[pl.ds(off, CH_H)], idx_v)

        @pl.loop(0, CH_H // 16)
        def _(j):
            iv = idx_v[pl.ds(j * 16, 16)]
            plsc.addupdate_scatter(hist_v, [iv], ones16)

    pltpu.sync_copy(hist_v, stg_sh.at[sid])
    plsc.subcore_barrier()

    pltpu.sync_copy(
        stg_sh.at[pl.ds(0, NS), pl.ds(sid * ROWS_PER_TILE, ROWS_PER_TILE)],
        rbuf)

    @pl.loop(0, ROWS_PER_TILE // 16)
    def _(j):
        acc = rbuf[0, pl.ds(j * 16, 16)]
        def add_row(t, a):
            return a + rbuf[t, pl.ds(j * 16, 16)]
        acc = lax.fori_loop(1, NS, add_row, acc)
        obuf.at[pl.ds(j * 16, 16)][...] = acc

    pltpu.sync_copy(
        obuf, deg_hbm.at[pl.ds(cid * N_PAD + sid * ROWS_PER_TILE,
                               ROWS_PER_TILE)])


N_BLKS = EDGES_PER_TILE // CHUNK
G_CH = 25
G_IDX = G_CH * CHUNK
N_GRP = N_BLKS // G_CH


@functools.partial(
    pl.kernel,
    out_type=jax.ShapeDtypeStruct((NC, N_PAD, D_FEAT), jnp.float32),
    mesh=_mesh,
    scratch_types=[
        pltpu.VMEM_SHARED((N_PAD, D_FEAT), jnp.float32),
        pltpu.VMEM((G_IDX,), jnp.int32),
        pltpu.VMEM((G_IDX,), jnp.int32),
        pltpu.VMEM((CHUNK,), jnp.int32),
        pltpu.VMEM((CHUNK,), jnp.int32),
        pltpu.VMEM((CHUNK,), jnp.int32),
        pltpu.VMEM((CHUNK,), jnp.int32),
        pltpu.VMEM((CHUNK, D_FEAT), jnp.float32),
        pltpu.VMEM((CHUNK, D_FEAT), jnp.float32),
        pltpu.VMEM((ZCH, D_FEAT), jnp.float32),
        pltpu.SemaphoreType.DMA,
        pltpu.SemaphoreType.DMA,
    ],
)
def _agg_kernel(fs_hbm, edges_hbm, acc_hbm, acc_sh, sg_v, dg_v, sidx_a,
                didx_a, sidx_b, didx_b, rows_a, rows_b, zbuf, sem_a, sem_b):
    cid = lax.axis_index("c")
    sid = lax.axis_index("s")
    wid = cid * NS + sid
    _zero_fill(zbuf, ZCH, D_FEAT)

    @pl.loop(0, ROWS_PER_TILE // ZCH)
    def _(k):
        pltpu.sync_copy(zbuf,
                        acc_sh.at[pl.ds(sid * ROWS_PER_TILE + k * ZCH, ZCH)])

    plsc.subcore_barrier()

    base = wid * EDGES_PER_TILE

    def copy_idx(slot, sidx, didx):
        @pl.loop(0, CHUNK // 16)
        def _(i):
            sidx.at[pl.ds(i * 16, 16)][...] = sg_v[pl.ds(slot * CHUNK + i * 16, 16)]
            didx.at[pl.ds(i * 16, 16)][...] = dg_v[pl.ds(slot * CHUNK + i * 16, 16)]

    @pl.loop(0, N_GRP)
    def _(g):
        goff = base + g * G_IDX
        pltpu.sync_copy(edges_hbm.at[pl.ds(goff, G_IDX)], sg_v)
        pltpu.sync_copy(edges_hbm.at[pl.ds(N_EDGES + goff, G_IDX)], dg_v)

        copy_idx(0, sidx_a, didx_a)
        pltpu.async_copy(fs_hbm.at[sidx_a], rows_a, sem_a)

        @pl.loop(0, (G_CH - 1) // 2)
        def _(k):
            copy_idx(2 * k + 1, sidx_b, didx_b)
            pltpu.async_copy(fs_hbm.at[sidx_b], rows_b, sem_b)
            pltpu.make_async_copy(fs_hbm.at[sidx_a], rows_a, sem_a).wait()
            pltpu.sync_copy(rows_a, acc_sh.at[didx_a], add=True)
            copy_idx(2 * k + 2, sidx_a, didx_a)
            pltpu.async_copy(fs_hbm.at[sidx_a], rows_a, sem_a)
            pltpu.make_async_copy(fs_hbm.at[sidx_b], rows_b, sem_b).wait()
            pltpu.sync_copy(rows_b, acc_sh.at[didx_b], add=True)

        pltpu.make_async_copy(fs_hbm.at[sidx_a], rows_a, sem_a).wait()
        pltpu.sync_copy(rows_a, acc_sh.at[didx_a], add=True)

    plsc.subcore_barrier()
    pltpu.sync_copy(
        acc_sh.at[pl.ds(sid * ROWS_PER_TILE, ROWS_PER_TILE)],
        acc_hbm.at[cid, pl.ds(sid * ROWS_PER_TILE, ROWS_PER_TILE)])


def _fs_body(feat_ref, deg_ref, fs_ref):
    sn = lax.rsqrt(jnp.maximum(deg_ref[...], 1.0))
    fs_ref[...] = feat_ref[...] * sn


_fs_call = pl.pallas_call(
    _fs_body,
    out_shape=jax.ShapeDtypeStruct((N_NODES, D_FEAT), jnp.float32),
)


FIN_B = 1000


def _fin_body(pp_ref, deg_ref, feat_ref, w_ref, b_ref, x_ref, steps_ref,
              rem_ref):
    dn = lax.rsqrt(jnp.maximum(deg_ref[...], 1.0))
    prop = (pp_ref[0] + pp_ref[1]) * dn
    lg = jnp.dot(prop, w_ref[...])[:, 0:1] + b_ref[...]
    h = jax.nn.sigmoid(lg)
    steps = jnp.ones((FIN_B, 1), jnp.float32)
    sum_h = jnp.zeros((FIN_B, 1), jnp.float32)
    cont = jnp.ones((FIN_B, 1), dtype=jnp.bool_)
    acoef = jnp.zeros((FIN_B, 1), jnp.float32)
    bcoef = jnp.zeros((FIN_B, 1), jnp.float32)
    for k in range(NITER):
        cf = cont.astype(jnp.float32)
        pm = (sum_h + h < 0.99) & cont
        pf = pm.astype(jnp.float32)
        steps = steps + pf
        sum_h = sum_h + pf * h
        cond = pm & (steps < float(NITER))
        p = jnp.where(cond, sum_h, 1.0 - sum_h)
        if k == 0:
            acoef = acoef + p * cf
            bcoef = bcoef + (1.0 - p) * cf
        else:
            acoef = acoef + cf
        cont = cont & pm
    x_ref[...] = (acoef * prop + bcoef * feat_ref[...]) / steps
    steps_ref[...] = steps
    rem_ref[...] = 1.0 - sum_h


_fin_call = pl.pallas_call(
    _fin_body,
    grid=(N_NODES // FIN_B,),
    in_specs=[
        pl.BlockSpec((2, FIN_B, D_FEAT), lambda i: (0, i, 0)),
        pl.BlockSpec((FIN_B, 1), lambda i: (i, 0)),
        pl.BlockSpec((FIN_B, D_FEAT), lambda i: (i, 0)),
        pl.BlockSpec((D_FEAT, D_FEAT), lambda i: (0, 0)),
        pl.BlockSpec((1, 1), lambda i: (0, 0)),
    ],
    out_specs=(
        pl.BlockSpec((FIN_B, D_FEAT), lambda i: (i, 0)),
        pl.BlockSpec((FIN_B, 1), lambda i: (i, 0)),
        pl.BlockSpec((FIN_B, 1), lambda i: (i, 0)),
    ),
    out_shape=(
        jax.ShapeDtypeStruct((N_NODES, D_FEAT), jnp.float32),
        jax.ShapeDtypeStruct((N_NODES, 1), jnp.float32),
        jax.ShapeDtypeStruct((N_NODES, 1), jnp.float32),
    ),
)


@jax.jit
def kernel(feature, edge_index, W_halt, b_halt):
    edges_flat = edge_index.reshape(-1)
    degs = _deg_kernel(edges_flat)
    out_deg = degs[0:N_NODES, None]
    in_deg = degs[N_PAD:N_PAD + N_NODES, None]
    fs = _fs_call(feature, out_deg)
    pp = _agg_kernel(fs, edges_flat)
    w128 = jnp.zeros((D_FEAT, D_FEAT), jnp.float32).at[:, 0].set(W_halt[0])
    x, steps2, rem2 = _fin_call(pp, in_deg, feature, w128,
                                b_halt.reshape(1, 1))
    return (x, steps2.reshape(-1), rem2.reshape(-1))


# speedup vs baseline: 11.1982x; 1.0270x over previous
import dataclasses
import functools

import jax
import jax.numpy as jnp
from jax import lax
from jax.experimental import pallas as pl
from jax.experimental.pallas import tpu as pltpu
from jax.experimental.pallas import tpu_sc as plsc

N_NODES = 10000
N_EDGES = 320000
D_FEAT = 128
NITER = 10

NC, NS = 2, 16
NTILES = NC * NS
EDGES_PER_TILE = N_EDGES // NTILES
EDGES_PER_SUB = N_EDGES // NS
CHUNK = 80
N_PAD = 10240
ROWS_PER_TILE = N_PAD // NS
ZCH = 128
DEG_W = 16

_mesh = plsc.VectorSubcoreMesh(
    core_axis_name="c", subcore_axis_name="s", num_cores=NC, num_subcores=NS)


def _zero_fill(buf, rows, width):
    @pl.loop(0, rows)
    def _(i):
        @pl.loop(0, width // 16)
        def _(j):
            buf.at[pl.ds(i, 1), pl.ds(j * 16, 16)][...] = (
                jnp.zeros((1, 16), jnp.float32))


CH_H = 2000
N_HCH = EDGES_PER_SUB // CH_H

_cp_no_layout = pltpu.CompilerParams()
if "needs_layout_passes" in pltpu.CompilerParams.__dataclass_fields__:
    _cp_no_layout = dataclasses.replace(_cp_no_layout,
                                        needs_layout_passes=False)


@functools.partial(
    pl.kernel,
    compiler_params=_cp_no_layout,
    out_type=jax.ShapeDtypeStruct((2 * N_PAD,), jnp.float32),
    mesh=_mesh,
    scratch_types=[
        pltpu.VMEM_SHARED((NS, N_PAD), jnp.float32),
        pltpu.VMEM((N_PAD,), jnp.float32),
        pltpu.VMEM((CH_H,), jnp.int32),
        pltpu.VMEM((NS, ROWS_PER_TILE), jnp.float32),
        pltpu.VMEM((ROWS_PER_TILE,), jnp.float32),
    ],
)
def _deg_kernel(edges_hbm, deg_hbm, stg_sh, hist_v, idx_v, rbuf, obuf):
    cid = lax.axis_index("c")
    sid = lax.axis_index("s")
    ones16 = jnp.ones((16,), jnp.float32)

    @pl.loop(0, N_PAD // 16)
    def _(i):
        hist_v.at[pl.ds(i * 16, 16)][...] = jnp.zeros((16,), jnp.float32)

    @pl.loop(0, N_HCH)
    def _(k):
        off = cid * N_EDGES + sid * EDGES_PER_SUB + k * CH_H
        pltpu.sync_copy(edges_hbm.at[pl.ds(off, CH_H)], idx_v)

        @pl.loop(0, CH_H // 16)
        def _(j):
            iv = idx_v[pl.ds(j * 16, 16)]
            plsc.addupdate_scatter(hist_v, [iv], ones16)

    pltpu.sync_copy(hist_v, stg_sh.at[sid])
    plsc.subcore_barrier()

    pltpu.sync_copy(
        stg_sh.at[pl.ds(0, NS), pl.ds(sid * ROWS_PER_TILE, ROWS_PER_TILE)],
        rbuf)

    @pl.loop(0, ROWS_PER_TILE // 16)
    def _(j):
        acc = rbuf[0, pl.ds(j * 16, 16)]
        def add_row(t, a):
            return a + rbuf[t, pl.ds(j * 16, 16)]
        acc = lax.fori_loop(1, NS, add_row, acc)
        obuf.at[pl.ds(j * 16, 16)][...] = acc

    pltpu.sync_copy(
        obuf, deg_hbm.at[pl.ds(cid * N_PAD + sid * ROWS_PER_TILE,
                               ROWS_PER_TILE)])


CHUNK2 = 128
N_BLKS = EDGES_PER_TILE // CHUNK2
TAIL = EDGES_PER_TILE - N_BLKS * CHUNK2
G_CH = 13
G_IDX = G_CH * CHUNK2
N_GRP = N_BLKS // G_CH
AZCH = 64


@functools.partial(
    pl.kernel,
    out_type=jax.ShapeDtypeStruct((NC, N_PAD, D_FEAT), jnp.float32),
    mesh=_mesh,
    scratch_types=[
        pltpu.VMEM_SHARED((N_PAD, D_FEAT), jnp.float32),
        pltpu.VMEM((G_IDX,), jnp.int32),
        pltpu.VMEM((G_IDX,), jnp.int32),
        pltpu.VMEM((CHUNK2,), jnp.int32),
        pltpu.VMEM((CHUNK2,), jnp.int32),
        pltpu.VMEM((CHUNK2,), jnp.int32),
        pltpu.VMEM((CHUNK2,), jnp.int32),
        pltpu.VMEM((TAIL,), jnp.int32),
        pltpu.VMEM((TAIL,), jnp.int32),
        pltpu.VMEM((CHUNK2, D_FEAT), jnp.float32),
        pltpu.VMEM((CHUNK2, D_FEAT), jnp.float32),
        pltpu.VMEM((TAIL, D_FEAT), jnp.float32),
        pltpu.VMEM((AZCH, D_FEAT), jnp.float32),
        pltpu.SemaphoreType.DMA,
        pltpu.SemaphoreType.DMA,
    ],
)
def _agg_kernel(fs_hbm, edges_hbm, acc_hbm, acc_sh, sg_v, dg_v, sidx_a,
                didx_a, sidx_b, didx_b, sidx_t, didx_t, rows_a, rows_b,
                rows_t, zbuf, sem_a, sem_b):
    cid = lax.axis_index("c")
    sid = lax.axis_index("s")
    wid = cid * NS + sid
    _zero_fill(zbuf, AZCH, D_FEAT)

    @pl.loop(0, ROWS_PER_TILE // AZCH)
    def _(k):
        pltpu.sync_copy(zbuf,
                        acc_sh.at[pl.ds(sid * ROWS_PER_TILE + k * AZCH, AZCH)])

    plsc.subcore_barrier()

    base = wid * EDGES_PER_TILE

    def copy_idx(slot, sidx, didx):
        @pl.loop(0, CHUNK2 // 16)
        def _(i):
            sidx.at[pl.ds(i * 16, 16)][...] = sg_v[pl.ds(slot * CHUNK2 + i * 16, 16)]
            didx.at[pl.ds(i * 16, 16)][...] = dg_v[pl.ds(slot * CHUNK2 + i * 16, 16)]

    @pl.loop(0, N_GRP)
    def _(g):
        goff = base + g * G_IDX
        pltpu.sync_copy(edges_hbm.at[pl.ds(goff, G_IDX)], sg_v)
        pltpu.sync_copy(edges_hbm.at[pl.ds(N_EDGES + goff, G_IDX)], dg_v)

        copy_idx(0, sidx_a, didx_a)
        pltpu.async_copy(fs_hbm.at[sidx_a], rows_a, sem_a)

        @pl.loop(0, (G_CH - 1) // 2)
        def _(k):
            copy_idx(2 * k + 1, sidx_b, didx_b)
            pltpu.async_copy(fs_hbm.at[sidx_b], rows_b, sem_b)
            pltpu.make_async_copy(fs_hbm.at[sidx_a], rows_a, sem_a).wait()
            pltpu.sync_copy(rows_a, acc_sh.at[didx_a], add=True)
            copy_idx(2 * k + 2, sidx_a, didx_a)
            pltpu.async_copy(fs_hbm.at[sidx_a], rows_a, sem_a)
            pltpu.make_async_copy(fs_hbm.at[sidx_b], rows_b, sem_b).wait()
            pltpu.sync_copy(rows_b, acc_sh.at[didx_b], add=True)

        pltpu.make_async_copy(fs_hbm.at[sidx_a], rows_a, sem_a).wait()
        pltpu.sync_copy(rows_a, acc_sh.at[didx_a], add=True)

    toff = base + N_BLKS * CHUNK2
    pltpu.sync_copy(edges_hbm.at[pl.ds(toff, TAIL)], sidx_t)
    pltpu.sync_copy(edges_hbm.at[pl.ds(N_EDGES + toff, TAIL)], didx_t)
    pltpu.async_copy(fs_hbm.at[sidx_t], rows_t, sem_a).wait()
    pltpu.sync_copy(rows_t, acc_sh.at[didx_t], add=True)

    plsc.subcore_barrier()
    pltpu.sync_copy(
        acc_sh.at[pl.ds(sid * ROWS_PER_TILE, ROWS_PER_TILE)],
        acc_hbm.at[cid, pl.ds(sid * ROWS_PER_TILE, ROWS_PER_TILE)])


def _fs_body(feat_ref, deg_ref, fs_ref):
    sn = lax.rsqrt(jnp.maximum(deg_ref[...], 1.0))
    fs_ref[...] = feat_ref[...] * sn


_fs_call = pl.pallas_call(
    _fs_body,
    out_shape=jax.ShapeDtypeStruct((N_NODES, D_FEAT), jnp.float32),
)


FIN_B = 1000


def _fin_body(pp_ref, deg_ref, feat_ref, w_ref, b_ref, x_ref, steps_ref,
              rem_ref):
    dn = lax.rsqrt(jnp.maximum(deg_ref[...], 1.0))
    prop = (pp_ref[0] + pp_ref[1]) * dn
    lg = jnp.dot(prop, w_ref[...])[:, 0:1] + b_ref[...]
    h = jax.nn.sigmoid(lg)
    steps = jnp.ones((FIN_B, 1), jnp.float32)
    sum_h = jnp.zeros((FIN_B, 1), jnp.float32)
    cont = jnp.ones((FIN_B, 1), dtype=jnp.bool_)
    acoef = jnp.zeros((FIN_B, 1), jnp.float32)
    bcoef = jnp.zeros((FIN_B, 1), jnp.float32)
    for k in range(NITER):
        cf = cont.astype(jnp.float32)
        pm = (sum_h + h < 0.99) & cont
        pf = pm.astype(jnp.float32)
        steps = steps + pf
        sum_h = sum_h + pf * h
        cond = pm & (steps < float(NITER))
        p = jnp.where(cond, sum_h, 1.0 - sum_h)
        if k == 0:
            acoef = acoef + p * cf
            bcoef = bcoef + (1.0 - p) * cf
        else:
            acoef = acoef + cf
        cont = cont & pm
    x_ref[...] = (acoef * prop + bcoef * feat_ref[...]) / steps
    steps_ref[...] = steps
    rem_ref[...] = 1.0 - sum_h


_fin_call = pl.pallas_call(
    _fin_body,
    grid=(N_NODES // FIN_B,),
    in_specs=[
        pl.BlockSpec((2, FIN_B, D_FEAT), lambda i: (0, i, 0)),
        pl.BlockSpec((FIN_B, 1), lambda i: (i, 0)),
        pl.BlockSpec((FIN_B, D_FEAT), lambda i: (i, 0)),
        pl.BlockSpec((D_FEAT, D_FEAT), lambda i: (0, 0)),
        pl.BlockSpec((1, 1), lambda i: (0, 0)),
    ],
    out_specs=(
        pl.BlockSpec((FIN_B, D_FEAT), lambda i: (i, 0)),
        pl.BlockSpec((FIN_B, 1), lambda i: (i, 0)),
        pl.BlockSpec((FIN_B, 1), lambda i: (i, 0)),
    ),
    out_shape=(
        jax.ShapeDtypeStruct((N_NODES, D_FEAT), jnp.float32),
        jax.ShapeDtypeStruct((N_NODES, 1), jnp.float32),
        jax.ShapeDtypeStruct((N_NODES, 1), jnp.float32),
    ),
)


@jax.jit
def kernel(feature, edge_index, W_halt, b_halt):
    edges_flat = edge_index.reshape(-1)
    degs = _deg_kernel(edges_flat)
    out_deg = degs[0:N_NODES, None]
    in_deg = degs[N_PAD:N_PAD + N_NODES, None]
    fs = _fs_call(feature, out_deg)
    pp = _agg_kernel(fs, edges_flat)
    w128 = jnp.zeros((D_FEAT, D_FEAT), jnp.float32).at[:, 0].set(W_halt[0])
    x, steps2, rem2 = _fin_call(pp, in_deg, feature, w128,
                                b_halt.reshape(1, 1))
    return (x, steps2.reshape(-1), rem2.reshape(-1))
